# Initial kernel scaffold; baseline (speedup 1.0000x reference)
import jax
import jax.numpy as jnp
from jax.experimental import pallas as pl


def kernel(x, adj_mat, W1, a1_l, a1_r, W2, a2_l, a2_r):
    raise NotImplementedError("write your pallas kernel here")


# speedup vs baseline: 1.5711x; 1.5711x over previous
import jax
import jax.numpy as jnp
from jax.experimental import pallas as pl

N = 2048
H1 = 8
F1 = 32
D1 = H1 * F1
F2 = 32
BI = 256


def _leaky(x):
    return jnp.where(x >= 0, x, 0.2 * x)


def _proj1_body(x_ref, w_ref, al_ref, ar_ref, g_ref, el_ref, er_ref):
    g = jnp.dot(x_ref[...], w_ref[...], preferred_element_type=jnp.float32)
    g_ref[...] = g
    el_ref[...] = jnp.dot(g, al_ref[...], preferred_element_type=jnp.float32)
    er_ref[...] = jnp.dot(g, ar_ref[...], preferred_element_type=jnp.float32)


def _attn1_body(adj_ref, g_ref, el_ref, er_ref, w2_ref, a2l_ref, a2r_ref,
                g2_ref, el2_ref, er2_ref):
    mask = adj_ref[...]
    g = g_ref[...]
    el = el_ref[...]
    ert = er_ref[...].T
    ermax = jnp.max(ert, axis=1, keepdims=True)
    outs = []
    for h in range(H1):
        s = _leaky(el[:, h:h + 1] + ert[h:h + 1, :])
        bound = _leaky(el[:, h:h + 1] + ermax[h, 0])
        w = jnp.where(mask, jnp.exp(s - bound), 0.0)
        denom = jnp.sum(w, axis=1, keepdims=True)
        gh = g[:, h * F1:(h + 1) * F1]
        num = jnp.dot(w, gh, preferred_element_type=jnp.float32)
        gmean = jnp.mean(gh, axis=0, keepdims=True)
        outs.append(jnp.where(denom > 0, num / denom, gmean))
    hcat = jnp.concatenate(outs, axis=1)
    hact = jnp.where(hcat > 0, hcat, jnp.exp(hcat) - 1.0)
    g2 = jnp.dot(hact, w2_ref[...], preferred_element_type=jnp.float32)
    g2_ref[...] = g2
    el2_ref[...] = jnp.dot(g2, a2l_ref[...], preferred_element_type=jnp.float32)
    er2_ref[...] = jnp.dot(g2, a2r_ref[...], preferred_element_type=jnp.float32)


def _attn2_body(adj_ref, g2_ref, el2_ref, er2_ref, out_ref):
    mask = adj_ref[...]
    g2 = g2_ref[...]
    el = el2_ref[...]
    er = er2_ref[...]
    ert = er.T
    ermax = jnp.max(er)
    s = _leaky(el + ert)
    bound = _leaky(el + ermax)
    w = jnp.where(mask, jnp.exp(s - bound), 0.0)
    denom = jnp.sum(w, axis=1, keepdims=True)
    num = jnp.dot(w, g2, preferred_element_type=jnp.float32)
    gmean = jnp.mean(g2, axis=0, keepdims=True)
    out_ref[...] = jnp.where(denom > 0, num / denom, gmean)


def kernel(x, adj_mat, W1, a1_l, a1_r, W2, a2_l, a2_r):
    adj = adj_mat.reshape(N, N)
    eye = jnp.eye(H1, dtype=jnp.float32)
    al1 = (eye[:, None, :] * a1_l[None, :, None]).reshape(D1, H1)
    ar1 = (eye[:, None, :] * a1_r[None, :, None]).reshape(D1, H1)
    a2l = a2_l.reshape(F2, 1)
    a2r = a2_r.reshape(F2, 1)

    nb = N // BI
    g1, el1, er1 = pl.pallas_call(
        _proj1_body,
        grid=(nb,),
        in_specs=[
            pl.BlockSpec((BI, x.shape[1]), lambda i: (i, 0)),
            pl.BlockSpec(W1.shape, lambda i: (0, 0)),
            pl.BlockSpec((D1, H1), lambda i: (0, 0)),
            pl.BlockSpec((D1, H1), lambda i: (0, 0)),
        ],
        out_specs=[
            pl.BlockSpec((BI, D1), lambda i: (i, 0)),
            pl.BlockSpec((BI, H1), lambda i: (i, 0)),
            pl.BlockSpec((BI, H1), lambda i: (i, 0)),
        ],
        out_shape=[
            jax.ShapeDtypeStruct((N, D1), jnp.float32),
            jax.ShapeDtypeStruct((N, H1), jnp.float32),
            jax.ShapeDtypeStruct((N, H1), jnp.float32),
        ],
    )(x, W1, al1, ar1)

    g2, el2, er2 = pl.pallas_call(
        _attn1_body,
        grid=(nb,),
        in_specs=[
            pl.BlockSpec((BI, N), lambda i: (i, 0)),
            pl.BlockSpec((N, D1), lambda i: (0, 0)),
            pl.BlockSpec((BI, H1), lambda i: (i, 0)),
            pl.BlockSpec((N, H1), lambda i: (0, 0)),
            pl.BlockSpec(W2.shape, lambda i: (0, 0)),
            pl.BlockSpec((F2, 1), lambda i: (0, 0)),
            pl.BlockSpec((F2, 1), lambda i: (0, 0)),
        ],
        out_specs=[
            pl.BlockSpec((BI, F2), lambda i: (i, 0)),
            pl.BlockSpec((BI, 1), lambda i: (i, 0)),
            pl.BlockSpec((BI, 1), lambda i: (i, 0)),
        ],
        out_shape=[
            jax.ShapeDtypeStruct((N, F2), jnp.float32),
            jax.ShapeDtypeStruct((N, 1), jnp.float32),
            jax.ShapeDtypeStruct((N, 1), jnp.float32),
        ],
    )(adj, g1, el1, er1, W2, a2l, a2r)

    out = pl.pallas_call(
        _attn2_body,
        grid=(nb,),
        in_specs=[
            pl.BlockSpec((BI, N), lambda i: (i, 0)),
            pl.BlockSpec((N, F2), lambda i: (0, 0)),
            pl.BlockSpec((BI, 1), lambda i: (i, 0)),
            pl.BlockSpec((N, 1), lambda i: (0, 0)),
        ],
        out_specs=pl.BlockSpec((BI, F2), lambda i: (i, 0)),
        out_shape=jax.ShapeDtypeStruct((N, F2), jnp.float32),
    )(adj, g2, el2, er2)
    return out


# speedup vs baseline: 1.8998x; 1.2092x over previous
import jax
import jax.numpy as jnp
from jax.experimental import pallas as pl

N = 2048
H1 = 8
F1 = 32
D1 = H1 * F1
F2 = 32
BI = 256


def _leaky(x):
    return jnp.where(x >= 0, x, 0.2 * x)


def _proj1_body(x_ref, w_ref, al_ref, ar_ref, g_ref, el_ref, er_ref):
    g = jnp.dot(x_ref[...], w_ref[...], preferred_element_type=jnp.float32)
    g_ref[...] = g
    el_ref[...] = jnp.dot(g, al_ref[...], preferred_element_type=jnp.float32)
    er_ref[...] = jnp.dot(g, ar_ref[...], preferred_element_type=jnp.float32)


def _attn1_body(adj_ref, g_ref, el_ref, er_ref, w2_ref, a2l_ref, a2r_ref,
                g2_ref, el2_ref, er2_ref):
    maskf = adj_ref[...].astype(jnp.float32)
    g = g_ref[...]
    el = el_ref[...]
    ert = er_ref[...].T
    ermax = jnp.max(ert, axis=1, keepdims=True)
    outs = []
    for h in range(H1):
        em = ermax[h:h + 1, :]
        x = el[:, h:h + 1] + em
        bound = _leaky(x)
        a = jnp.exp(x - bound)
        c = jnp.exp(0.2 * x - bound)
        b = jnp.exp(ert[h:h + 1, :] - em)
        d = jnp.exp(0.2 * (ert[h:h + 1, :] - em))
        w = jnp.maximum(a * b, c * d) * maskf
        denom = jnp.sum(w, axis=1, keepdims=True)
        gh = g[:, h * F1:(h + 1) * F1]
        num = jnp.dot(w, gh, preferred_element_type=jnp.float32)
        gmean = jnp.mean(gh, axis=0, keepdims=True)
        outs.append(jnp.where(denom > 0, num / denom, gmean))
    hcat = jnp.concatenate(outs, axis=1)
    hact = jnp.where(hcat > 0, hcat, jnp.exp(hcat) - 1.0)
    g2 = jnp.dot(hact, w2_ref[...], preferred_element_type=jnp.float32)
    g2_ref[...] = g2
    el2_ref[...] = jnp.dot(g2, a2l_ref[...], preferred_element_type=jnp.float32)
    er2_ref[...] = jnp.dot(g2, a2r_ref[...], preferred_element_type=jnp.float32)


def _attn2_body(adj_ref, g2_ref, el2_ref, er2_ref, out_ref):
    maskf = adj_ref[...].astype(jnp.float32)
    g2 = g2_ref[...]
    el = el2_ref[...]
    ert = er2_ref[...].T
    em = jnp.max(ert, axis=1, keepdims=True)
    x = el + em
    bound = _leaky(x)
    a = jnp.exp(x - bound)
    c = jnp.exp(0.2 * x - bound)
    b = jnp.exp(ert - em)
    d = jnp.exp(0.2 * (ert - em))
    w = jnp.maximum(a * b, c * d) * maskf
    denom = jnp.sum(w, axis=1, keepdims=True)
    num = jnp.dot(w, g2, preferred_element_type=jnp.float32)
    gmean = jnp.mean(g2, axis=0, keepdims=True)
    out_ref[...] = jnp.where(denom > 0, num / denom, gmean)


def kernel(x, adj_mat, W1, a1_l, a1_r, W2, a2_l, a2_r):
    adj = adj_mat.reshape(N, N)
    eye = jnp.eye(H1, dtype=jnp.float32)
    al1 = (eye[:, None, :] * a1_l[None, :, None]).reshape(D1, H1)
    ar1 = (eye[:, None, :] * a1_r[None, :, None]).reshape(D1, H1)
    a2l = a2_l.reshape(F2, 1)
    a2r = a2_r.reshape(F2, 1)

    nb = N // BI
    g1, el1, er1 = pl.pallas_call(
        _proj1_body,
        grid=(nb,),
        in_specs=[
            pl.BlockSpec((BI, x.shape[1]), lambda i: (i, 0)),
            pl.BlockSpec(W1.shape, lambda i: (0, 0)),
            pl.BlockSpec((D1, H1), lambda i: (0, 0)),
            pl.BlockSpec((D1, H1), lambda i: (0, 0)),
        ],
        out_specs=[
            pl.BlockSpec((BI, D1), lambda i: (i, 0)),
            pl.BlockSpec((BI, H1), lambda i: (i, 0)),
            pl.BlockSpec((BI, H1), lambda i: (i, 0)),
        ],
        out_shape=[
            jax.ShapeDtypeStruct((N, D1), jnp.float32),
            jax.ShapeDtypeStruct((N, H1), jnp.float32),
            jax.ShapeDtypeStruct((N, H1), jnp.float32),
        ],
    )(x, W1, al1, ar1)

    g2, el2, er2 = pl.pallas_call(
        _attn1_body,
        grid=(nb,),
        in_specs=[
            pl.BlockSpec((BI, N), lambda i: (i, 0)),
            pl.BlockSpec((N, D1), lambda i: (0, 0)),
            pl.BlockSpec((BI, H1), lambda i: (i, 0)),
            pl.BlockSpec((N, H1), lambda i: (0, 0)),
            pl.BlockSpec(W2.shape, lambda i: (0, 0)),
            pl.BlockSpec((F2, 1), lambda i: (0, 0)),
            pl.BlockSpec((F2, 1), lambda i: (0, 0)),
        ],
        out_specs=[
            pl.BlockSpec((BI, F2), lambda i: (i, 0)),
            pl.BlockSpec((BI, 1), lambda i: (i, 0)),
            pl.BlockSpec((BI, 1), lambda i: (i, 0)),
        ],
        out_shape=[
            jax.ShapeDtypeStruct((N, F2), jnp.float32),
            jax.ShapeDtypeStruct((N, 1), jnp.float32),
            jax.ShapeDtypeStruct((N, 1), jnp.float32),
        ],
    )(adj, g1, el1, er1, W2, a2l, a2r)

    out = pl.pallas_call(
        _attn2_body,
        grid=(nb,),
        in_specs=[
            pl.BlockSpec((BI, N), lambda i: (i, 0)),
            pl.BlockSpec((N, F2), lambda i: (0, 0)),
            pl.BlockSpec((BI, 1), lambda i: (i, 0)),
            pl.BlockSpec((N, 1), lambda i: (0, 0)),
        ],
        out_specs=pl.BlockSpec((BI, F2), lambda i: (i, 0)),
        out_shape=jax.ShapeDtypeStruct((N, F2), jnp.float32),
    )(adj, g2, el2, er2)
    return out


# speedup vs baseline: 2.1154x; 1.1135x over previous
import jax
import jax.numpy as jnp
from jax.experimental import pallas as pl

N = 2048
H1 = 8
F1 = 32
D1 = H1 * F1
F2 = 32
BI = 256


def _leaky(x):
    return jnp.where(x >= 0, x, 0.2 * x)


def _proj1_body(x_ref, w_ref, al_ref, ar_ref,
                gb_ref, el_ref, ert_ref, gsum_ref):
    g = jnp.dot(x_ref[...], w_ref[...], preferred_element_type=jnp.float32)
    gb_ref[...] = g.astype(jnp.bfloat16)
    el_ref[...] = jnp.dot(g, al_ref[...], preferred_element_type=jnp.float32)
    er = jnp.dot(g, ar_ref[...], preferred_element_type=jnp.float32)
    ert_ref[...] = er.T
    colsum = jnp.sum(g, axis=0, keepdims=True)

    @pl.when(pl.program_id(0) == 0)
    def _init():
        gsum_ref[...] = colsum

    @pl.when(pl.program_id(0) != 0)
    def _acc():
        gsum_ref[...] += colsum


def _attn1_body(adj_ref, gb_ref, el_ref, ert_ref, gsum_ref,
                w2_ref, a2l_ref, a2r_ref,
                g2_ref, el2_ref, er2t_ref, gsum2_ref):
    maskf = adj_ref[...].astype(jnp.float32)
    gb = gb_ref[...]
    el = el_ref[...]
    ert = ert_ref[...]
    ermax = jnp.max(ert, axis=1, keepdims=True)
    bmat = jnp.exp(ert - ermax)
    dmat = jnp.exp(0.2 * (ert - ermax))
    gmean = gsum_ref[...] * (1.0 / N)
    outs = []
    for h in range(H1):
        x = el[:, h:h + 1] + ermax[h:h + 1, :]
        bound = _leaky(x)
        a = jnp.exp(x - bound)
        c = jnp.exp(0.2 * x - bound)
        w = jnp.maximum(a * bmat[h:h + 1, :],
                        c * dmat[h:h + 1, :]) * maskf
        denom = jnp.sum(w, axis=1, keepdims=True)
        num = jnp.dot(w.astype(jnp.bfloat16), gb[:, h * F1:(h + 1) * F1],
                      preferred_element_type=jnp.float32)
        outs.append(jnp.where(denom > 0, num / denom,
                              gmean[:, h * F1:(h + 1) * F1]))
    hcat = jnp.concatenate(outs, axis=1)
    hact = jnp.where(hcat > 0, hcat, jnp.exp(hcat) - 1.0)
    g2 = jnp.dot(hact, w2_ref[...], preferred_element_type=jnp.float32)
    g2_ref[...] = g2
    el2_ref[...] = jnp.dot(g2, a2l_ref[...], preferred_element_type=jnp.float32)
    er2 = jnp.dot(g2, a2r_ref[...], preferred_element_type=jnp.float32)
    er2t_ref[...] = er2.T
    colsum2 = jnp.sum(g2, axis=0, keepdims=True)

    @pl.when(pl.program_id(0) == 0)
    def _init():
        gsum2_ref[...] = colsum2

    @pl.when(pl.program_id(0) != 0)
    def _acc():
        gsum2_ref[...] += colsum2


def _attn2_body(adj_ref, g2_ref, el2_ref, er2t_ref, gsum2_ref, out_ref):
    maskf = adj_ref[...].astype(jnp.float32)
    g2 = g2_ref[...]
    el = el2_ref[...]
    ert = er2t_ref[...]
    em = jnp.max(ert, axis=1, keepdims=True)
    bvec = jnp.exp(ert - em)
    dvec = jnp.exp(0.2 * (ert - em))
    x = el + em
    bound = _leaky(x)
    a = jnp.exp(x - bound)
    c = jnp.exp(0.2 * x - bound)
    w = jnp.maximum(a * bvec, c * dvec) * maskf
    denom = jnp.sum(w, axis=1, keepdims=True)
    num = jnp.dot(w.astype(jnp.bfloat16), g2.astype(jnp.bfloat16),
                  preferred_element_type=jnp.float32)
    gmean = gsum2_ref[...] * (1.0 / N)
    out_ref[...] = jnp.where(denom > 0, num / denom, gmean)


def kernel(x, adj_mat, W1, a1_l, a1_r, W2, a2_l, a2_r):
    adj = adj_mat.reshape(N, N)
    eye = jnp.eye(H1, dtype=jnp.float32)
    al1 = (eye[:, None, :] * a1_l[None, :, None]).reshape(D1, H1)
    ar1 = (eye[:, None, :] * a1_r[None, :, None]).reshape(D1, H1)
    a2l = a2_l.reshape(F2, 1)
    a2r = a2_r.reshape(F2, 1)

    nb = N // BI
    gb1, el1, ert1, gsum1 = pl.pallas_call(
        _proj1_body,
        grid=(nb,),
        in_specs=[
            pl.BlockSpec((BI, x.shape[1]), lambda i: (i, 0)),
            pl.BlockSpec(W1.shape, lambda i: (0, 0)),
            pl.BlockSpec((D1, H1), lambda i: (0, 0)),
            pl.BlockSpec((D1, H1), lambda i: (0, 0)),
        ],
        out_specs=[
            pl.BlockSpec((BI, D1), lambda i: (i, 0)),
            pl.BlockSpec((BI, H1), lambda i: (i, 0)),
            pl.BlockSpec((H1, BI), lambda i: (0, i)),
            pl.BlockSpec((1, D1), lambda i: (0, 0)),
        ],
        out_shape=[
            jax.ShapeDtypeStruct((N, D1), jnp.bfloat16),
            jax.ShapeDtypeStruct((N, H1), jnp.float32),
            jax.ShapeDtypeStruct((H1, N), jnp.float32),
            jax.ShapeDtypeStruct((1, D1), jnp.float32),
        ],
    )(x, W1, al1, ar1)

    g2, el2, er2t, gsum2 = pl.pallas_call(
        _attn1_body,
        grid=(nb,),
        in_specs=[
            pl.BlockSpec((BI, N), lambda i: (i, 0)),
            pl.BlockSpec((N, D1), lambda i: (0, 0)),
            pl.BlockSpec((BI, H1), lambda i: (i, 0)),
            pl.BlockSpec((H1, N), lambda i: (0, 0)),
            pl.BlockSpec((1, D1), lambda i: (0, 0)),
            pl.BlockSpec(W2.shape, lambda i: (0, 0)),
            pl.BlockSpec((F2, 1), lambda i: (0, 0)),
            pl.BlockSpec((F2, 1), lambda i: (0, 0)),
        ],
        out_specs=[
            pl.BlockSpec((BI, F2), lambda i: (i, 0)),
            pl.BlockSpec((BI, 1), lambda i: (i, 0)),
            pl.BlockSpec((1, BI), lambda i: (0, i)),
            pl.BlockSpec((1, F2), lambda i: (0, 0)),
        ],
        out_shape=[
            jax.ShapeDtypeStruct((N, F2), jnp.float32),
            jax.ShapeDtypeStruct((N, 1), jnp.float32),
            jax.ShapeDtypeStruct((1, N), jnp.float32),
            jax.ShapeDtypeStruct((1, F2), jnp.float32),
        ],
    )(adj, gb1, el1, ert1, gsum1, W2, a2l, a2r)

    out = pl.pallas_call(
        _attn2_body,
        grid=(nb,),
        in_specs=[
            pl.BlockSpec((BI, N), lambda i: (i, 0)),
            pl.BlockSpec((N, F2), lambda i: (0, 0)),
            pl.BlockSpec((BI, 1), lambda i: (i, 0)),
            pl.BlockSpec((1, N), lambda i: (0, 0)),
            pl.BlockSpec((1, F2), lambda i: (0, 0)),
        ],
        out_specs=pl.BlockSpec((BI, F2), lambda i: (i, 0)),
        out_shape=jax.ShapeDtypeStruct((N, F2), jnp.float32),
    )(adj, g2, el2, er2t, gsum2)
    return out


# speedup vs baseline: 2.5748x; 1.2172x over previous
import jax
import jax.numpy as jnp
from jax.experimental import pallas as pl

N = 2048
H1 = 8
F1 = 32
D1 = H1 * F1
F2 = 32
BI = 256


def _leaky(x):
    return jnp.where(x >= 0, x, 0.2 * x)


def _proj1_body(x_ref, w_ref, al_ref, ar_ref,
                gb_ref, el_ref, ert_ref, gsum_ref):
    g = jnp.dot(x_ref[...], w_ref[...], preferred_element_type=jnp.float32)
    gb_ref[...] = g.astype(jnp.bfloat16)
    el_ref[...] = jnp.dot(g, al_ref[...], preferred_element_type=jnp.float32)
    er = jnp.dot(g, ar_ref[...], preferred_element_type=jnp.float32)
    ert_ref[...] = er.T
    colsum = jnp.sum(g, axis=0, keepdims=True)

    @pl.when(pl.program_id(0) == 0)
    def _init():
        gsum_ref[...] = colsum

    @pl.when(pl.program_id(0) != 0)
    def _acc():
        gsum_ref[...] += colsum


def _attn1_body(adj_ref, gb_ref, el_ref, ert_ref, gsum_ref,
                w2_ref, a2l_ref, a2r_ref,
                g2_ref, el2_ref, er2t_ref, gsum2_ref):
    maskb = adj_ref[...].astype(jnp.bfloat16)
    gb = gb_ref[...]
    el = el_ref[...]
    ert = ert_ref[...]
    ermax = jnp.max(ert, axis=1, keepdims=True)
    bmat = jnp.exp(ert - ermax).astype(jnp.bfloat16)
    dmat = jnp.exp(0.2 * (ert - ermax)).astype(jnp.bfloat16)
    gmean = gsum_ref[...] * (1.0 / N)
    ones = jnp.ones((N, 1), jnp.bfloat16)
    outs = []
    for h in range(H1):
        x = el[:, h:h + 1] + ermax[h:h + 1, :]
        bound = _leaky(x)
        a = jnp.exp(x - bound).astype(jnp.bfloat16)
        c = jnp.exp(0.2 * x - bound).astype(jnp.bfloat16)
        w = jnp.maximum(a * bmat[h:h + 1, :],
                        c * dmat[h:h + 1, :]) * maskb
        gbh = jnp.concatenate([gb[:, h * F1:(h + 1) * F1], ones], axis=1)
        r = jnp.dot(w, gbh, preferred_element_type=jnp.float32)
        num = r[:, :F1]
        denom = r[:, F1:F1 + 1]
        outs.append(jnp.where(denom > 0, num / denom,
                              gmean[:, h * F1:(h + 1) * F1]))
    hcat = jnp.concatenate(outs, axis=1)
    hact = jnp.where(hcat > 0, hcat, jnp.exp(hcat) - 1.0)
    g2 = jnp.dot(hact, w2_ref[...], preferred_element_type=jnp.float32)
    g2_ref[...] = g2
    el2_ref[...] = jnp.dot(g2, a2l_ref[...], preferred_element_type=jnp.float32)
    er2 = jnp.dot(g2, a2r_ref[...], preferred_element_type=jnp.float32)
    er2t_ref[...] = er2.T
    colsum2 = jnp.sum(g2, axis=0, keepdims=True)

    @pl.when(pl.program_id(0) == 0)
    def _init():
        gsum2_ref[...] = colsum2

    @pl.when(pl.program_id(0) != 0)
    def _acc():
        gsum2_ref[...] += colsum2


def _attn2_body(adj_ref, g2_ref, el2_ref, er2t_ref, gsum2_ref, out_ref):
    maskb = adj_ref[...].astype(jnp.bfloat16)
    g2 = g2_ref[...]
    el = el2_ref[...]
    ert = er2t_ref[...]
    em = jnp.max(ert, axis=1, keepdims=True)
    bvec = jnp.exp(ert - em).astype(jnp.bfloat16)
    dvec = jnp.exp(0.2 * (ert - em)).astype(jnp.bfloat16)
    x = el + em
    bound = _leaky(x)
    a = jnp.exp(x - bound).astype(jnp.bfloat16)
    c = jnp.exp(0.2 * x - bound).astype(jnp.bfloat16)
    w = jnp.maximum(a * bvec, c * dvec) * maskb
    gb2 = jnp.concatenate([g2.astype(jnp.bfloat16),
                           jnp.ones((N, 1), jnp.bfloat16)], axis=1)
    r = jnp.dot(w, gb2, preferred_element_type=jnp.float32)
    denom = r[:, F2:F2 + 1]
    gmean = gsum2_ref[...] * (1.0 / N)
    out_ref[...] = jnp.where(denom > 0, r[:, :F2] / denom, gmean)


def kernel(x, adj_mat, W1, a1_l, a1_r, W2, a2_l, a2_r):
    adj = adj_mat.reshape(N, N)
    eye = jnp.eye(H1, dtype=jnp.float32)
    al1 = (eye[:, None, :] * a1_l[None, :, None]).reshape(D1, H1)
    ar1 = (eye[:, None, :] * a1_r[None, :, None]).reshape(D1, H1)
    a2l = a2_l.reshape(F2, 1)
    a2r = a2_r.reshape(F2, 1)

    nb = N // BI
    gb1, el1, ert1, gsum1 = pl.pallas_call(
        _proj1_body,
        grid=(nb,),
        in_specs=[
            pl.BlockSpec((BI, x.shape[1]), lambda i: (i, 0)),
            pl.BlockSpec(W1.shape, lambda i: (0, 0)),
            pl.BlockSpec((D1, H1), lambda i: (0, 0)),
            pl.BlockSpec((D1, H1), lambda i: (0, 0)),
        ],
        out_specs=[
            pl.BlockSpec((BI, D1), lambda i: (i, 0)),
            pl.BlockSpec((BI, H1), lambda i: (i, 0)),
            pl.BlockSpec((H1, BI), lambda i: (0, i)),
            pl.BlockSpec((1, D1), lambda i: (0, 0)),
        ],
        out_shape=[
            jax.ShapeDtypeStruct((N, D1), jnp.bfloat16),
            jax.ShapeDtypeStruct((N, H1), jnp.float32),
            jax.ShapeDtypeStruct((H1, N), jnp.float32),
            jax.ShapeDtypeStruct((1, D1), jnp.float32),
        ],
    )(x, W1, al1, ar1)

    g2, el2, er2t, gsum2 = pl.pallas_call(
        _attn1_body,
        grid=(nb,),
        in_specs=[
            pl.BlockSpec((BI, N), lambda i: (i, 0)),
            pl.BlockSpec((N, D1), lambda i: (0, 0)),
            pl.BlockSpec((BI, H1), lambda i: (i, 0)),
            pl.BlockSpec((H1, N), lambda i: (0, 0)),
            pl.BlockSpec((1, D1), lambda i: (0, 0)),
            pl.BlockSpec(W2.shape, lambda i: (0, 0)),
            pl.BlockSpec((F2, 1), lambda i: (0, 0)),
            pl.BlockSpec((F2, 1), lambda i: (0, 0)),
        ],
        out_specs=[
            pl.BlockSpec((BI, F2), lambda i: (i, 0)),
            pl.BlockSpec((BI, 1), lambda i: (i, 0)),
            pl.BlockSpec((1, BI), lambda i: (0, i)),
            pl.BlockSpec((1, F2), lambda i: (0, 0)),
        ],
        out_shape=[
            jax.ShapeDtypeStruct((N, F2), jnp.float32),
            jax.ShapeDtypeStruct((N, 1), jnp.float32),
            jax.ShapeDtypeStruct((1, N), jnp.float32),
            jax.ShapeDtypeStruct((1, F2), jnp.float32),
        ],
    )(adj, gb1, el1, ert1, gsum1, W2, a2l, a2r)

    out = pl.pallas_call(
        _attn2_body,
        grid=(nb,),
        in_specs=[
            pl.BlockSpec((BI, N), lambda i: (i, 0)),
            pl.BlockSpec((N, F2), lambda i: (0, 0)),
            pl.BlockSpec((BI, 1), lambda i: (i, 0)),
            pl.BlockSpec((1, N), lambda i: (0, 0)),
            pl.BlockSpec((1, F2), lambda i: (0, 0)),
        ],
        out_specs=pl.BlockSpec((BI, F2), lambda i: (i, 0)),
        out_shape=jax.ShapeDtypeStruct((N, F2), jnp.float32),
    )(adj, g2, el2, er2t, gsum2)
    return out


# speedup vs baseline: 2.9244x; 1.1358x over previous
import jax
import jax.numpy as jnp
from jax.experimental import pallas as pl
from jax.experimental.pallas import tpu as pltpu

N = 2048
H1 = 8
F1 = 32
D1 = H1 * F1
F2 = 32
BI = 256


def _leaky(x):
    return jnp.where(x >= 0, x, 0.2 * x)


def _body(x_ref, adj_ref, w1_ref, al_ref, ar_ref, w2_ref, a2l_ref, a2r_ref,
          out_ref,
          gb_s, el_s, ert_s, gsum_s, g2b_s, el2_s, er2_s, gsum2_s):
    p = pl.program_id(0)
    i = pl.program_id(1)

    @pl.when((p == 0) & (i == 0))
    def _proj():
        g = jnp.dot(x_ref[...], w1_ref[...], preferred_element_type=jnp.float32)
        gb_s[...] = g.astype(jnp.bfloat16)
        el_s[...] = jnp.dot(g, al_ref[...], preferred_element_type=jnp.float32)
        er = jnp.dot(g, ar_ref[...], preferred_element_type=jnp.float32)
        ert_s[...] = er.T
        gsum_s[...] = jnp.sum(g, axis=0, keepdims=True)
        g2b_s[:, F2:] = jnp.ones((N, 1), jnp.bfloat16)

    @pl.when(p == 0)
    def _attn1():
        maskb = adj_ref[...].astype(jnp.bfloat16)
        gb = gb_s[...]
        el = el_s[pl.ds(i * BI, BI), :]
        ert = ert_s[...]
        ermax = jnp.max(ert, axis=1, keepdims=True)
        bmat = jnp.exp(ert - ermax).astype(jnp.bfloat16)
        dmat = jnp.exp(0.2 * (ert - ermax)).astype(jnp.bfloat16)
        gmean = gsum_s[...] * (1.0 / N)
        ones = jnp.ones((N, 1), jnp.bfloat16)
        outs = []
        for h in range(H1):
            x = el[:, h:h + 1] + ermax[h:h + 1, :]
            bound = _leaky(x)
            a = jnp.exp(x - bound).astype(jnp.bfloat16)
            c = jnp.exp(0.2 * x - bound).astype(jnp.bfloat16)
            w = jnp.maximum(a * bmat[h:h + 1, :],
                            c * dmat[h:h + 1, :]) * maskb
            gbh = jnp.concatenate([gb[:, h * F1:(h + 1) * F1], ones], axis=1)
            r = jnp.dot(w, gbh, preferred_element_type=jnp.float32)
            num = r[:, :F1]
            denom = r[:, F1:F1 + 1]
            outs.append(jnp.where(denom > 0, num / denom,
                                  gmean[:, h * F1:(h + 1) * F1]))
        hcat = jnp.concatenate(outs, axis=1)
        hact = jnp.where(hcat > 0, hcat, jnp.exp(hcat) - 1.0)
        g2 = jnp.dot(hact, w2_ref[...], preferred_element_type=jnp.float32)
        g2b_s[pl.ds(i * BI, BI), :F2] = g2.astype(jnp.bfloat16)
        el2_s[pl.ds(i * BI, BI), :] = jnp.dot(
            g2, a2l_ref[...], preferred_element_type=jnp.float32)
        er2_s[pl.ds(i * BI, BI), :] = jnp.dot(
            g2, a2r_ref[...], preferred_element_type=jnp.float32)
        colsum2 = jnp.sum(g2, axis=0, keepdims=True)

        @pl.when(i == 0)
        def _init():
            gsum2_s[...] = colsum2

        @pl.when(i != 0)
        def _acc():
            gsum2_s[...] += colsum2

    @pl.when(p == 1)
    def _attn2():
        maskb = adj_ref[...].astype(jnp.bfloat16)
        el = el2_s[pl.ds(i * BI, BI), :]
        ert = er2_s[...].T
        em = jnp.max(ert, axis=1, keepdims=True)
        bvec = jnp.exp(ert - em).astype(jnp.bfloat16)
        dvec = jnp.exp(0.2 * (ert - em)).astype(jnp.bfloat16)
        x = el + em
        bound = _leaky(x)
        a = jnp.exp(x - bound).astype(jnp.bfloat16)
        c = jnp.exp(0.2 * x - bound).astype(jnp.bfloat16)
        w = jnp.maximum(a * bvec, c * dvec) * maskb
        r = jnp.dot(w, g2b_s[...], preferred_element_type=jnp.float32)
        denom = r[:, F2:F2 + 1]
        gmean = gsum2_s[...] * (1.0 / N)
        out_ref[...] = jnp.where(denom > 0, r[:, :F2] / denom, gmean)


def kernel(x, adj_mat, W1, a1_l, a1_r, W2, a2_l, a2_r):
    adj = adj_mat.reshape(N, N)
    eye = jnp.eye(H1, dtype=jnp.float32)
    al1 = (eye[:, None, :] * a1_l[None, :, None]).reshape(D1, H1)
    ar1 = (eye[:, None, :] * a1_r[None, :, None]).reshape(D1, H1)
    a2l = a2_l.reshape(F2, 1)
    a2r = a2_r.reshape(F2, 1)

    nb = N // BI
    out = pl.pallas_call(
        _body,
        grid=(2, nb),
        in_specs=[
            pl.BlockSpec((N, x.shape[1]), lambda p, i: (0, 0)),
            pl.BlockSpec((BI, N), lambda p, i: (i, 0)),
            pl.BlockSpec(W1.shape, lambda p, i: (0, 0)),
            pl.BlockSpec((D1, H1), lambda p, i: (0, 0)),
            pl.BlockSpec((D1, H1), lambda p, i: (0, 0)),
            pl.BlockSpec(W2.shape, lambda p, i: (0, 0)),
            pl.BlockSpec((F2, 1), lambda p, i: (0, 0)),
            pl.BlockSpec((F2, 1), lambda p, i: (0, 0)),
        ],
        out_specs=pl.BlockSpec((BI, F2), lambda p, i: (i, 0)),
        out_shape=jax.ShapeDtypeStruct((N, F2), jnp.float32),
        scratch_shapes=[
            pltpu.VMEM((N, D1), jnp.bfloat16),
            pltpu.VMEM((N, H1), jnp.float32),
            pltpu.VMEM((H1, N), jnp.float32),
            pltpu.VMEM((1, D1), jnp.float32),
            pltpu.VMEM((N, F2 + 1), jnp.bfloat16),
            pltpu.VMEM((N, 1), jnp.float32),
            pltpu.VMEM((N, 1), jnp.float32),
            pltpu.VMEM((1, F2), jnp.float32),
        ],
    )(x, adj, W1, al1, ar1, W2, a2l, a2r)
    return out


# speedup vs baseline: 3.4819x; 1.1906x over previous
import jax
import jax.numpy as jnp
from jax.experimental import pallas as pl
from jax.experimental.pallas import tpu as pltpu

N = 2048
H1 = 8
F1 = 32
D1 = H1 * F1
F2 = 32
BI = 256


def _leaky(x):
    return jnp.where(x >= 0, x, 0.2 * x)


def _body(x_ref, adj_ref, w1_ref, al_ref, ar_ref, w2_ref, a2l_ref, a2r_ref,
          out_ref,
          gb_s, el_s, ermax_s, bmat_s, dmat_s, gsum_s,
          g2b_s, el2_s, er2_s, gsum2_s):
    p = pl.program_id(0)
    i = pl.program_id(1)

    @pl.when((p == 0) & (i == 0))
    def _proj():
        g = jnp.dot(x_ref[...], w1_ref[...], preferred_element_type=jnp.float32)
        gb_s[...] = g.astype(jnp.bfloat16)
        el_s[...] = jnp.dot(g, al_ref[...], preferred_element_type=jnp.float32)
        er = jnp.dot(g, ar_ref[...], preferred_element_type=jnp.float32)
        ert = er.T
        ermax = jnp.max(ert, axis=1, keepdims=True)
        ermax_s[...] = ermax
        bmat_s[...] = jnp.exp(ert - ermax).astype(jnp.bfloat16)
        dmat_s[...] = jnp.exp(0.2 * (ert - ermax)).astype(jnp.bfloat16)
        gsum_s[...] = jnp.sum(g, axis=0, keepdims=True)
        g2b_s[:, F2:] = jnp.ones((N, 1), jnp.bfloat16)

    @pl.when(p == 0)
    def _attn1():
        maskb = adj_ref[...].astype(jnp.bfloat16)
        gb = gb_s[...]
        el = el_s[pl.ds(i * BI, BI), :]
        ermax = ermax_s[...]
        gmean = gsum_s[...] * (1.0 / N)
        ones = jnp.ones((N, 1), jnp.bfloat16)
        outs = []
        for h in range(H1):
            x = el[:, h:h + 1] + ermax[h:h + 1, :]
            bound = _leaky(x)
            a = jnp.exp(x - bound).astype(jnp.bfloat16)
            c = jnp.exp(0.2 * x - bound).astype(jnp.bfloat16)
            w = jnp.maximum(a * bmat_s[h:h + 1, :],
                            c * dmat_s[h:h + 1, :]) * maskb
            gbh = jnp.concatenate([gb[:, h * F1:(h + 1) * F1], ones], axis=1)
            r = jnp.dot(w, gbh, preferred_element_type=jnp.float32)
            num = r[:, :F1]
            denom = r[:, F1:F1 + 1]
            outs.append(jnp.where(denom > 0, num / denom,
                                  gmean[:, h * F1:(h + 1) * F1]))
        hcat = jnp.concatenate(outs, axis=1)
        hact = jnp.where(hcat > 0, hcat, jnp.exp(hcat) - 1.0)
        g2 = jnp.dot(hact, w2_ref[...], preferred_element_type=jnp.float32)
        g2b_s[pl.ds(i * BI, BI), :F2] = g2.astype(jnp.bfloat16)
        el2_s[pl.ds(i * BI, BI), :] = jnp.dot(
            g2, a2l_ref[...], preferred_element_type=jnp.float32)
        er2_s[pl.ds(i * BI, BI), :] = jnp.dot(
            g2, a2r_ref[...], preferred_element_type=jnp.float32)
        colsum2 = jnp.sum(g2, axis=0, keepdims=True)

        @pl.when(i == 0)
        def _init():
            gsum2_s[...] = colsum2

        @pl.when(i != 0)
        def _acc():
            gsum2_s[...] += colsum2

    @pl.when(p == 1)
    def _attn2():
        maskb = adj_ref[...].astype(jnp.bfloat16)
        el = el2_s[pl.ds(i * BI, BI), :]
        ert = er2_s[...].T
        em = jnp.max(ert, axis=1, keepdims=True)
        bvec = jnp.exp(ert - em).astype(jnp.bfloat16)
        dvec = jnp.exp(0.2 * (ert - em)).astype(jnp.bfloat16)
        x = el + em
        bound = _leaky(x)
        a = jnp.exp(x - bound).astype(jnp.bfloat16)
        c = jnp.exp(0.2 * x - bound).astype(jnp.bfloat16)
        w = jnp.maximum(a * bvec, c * dvec) * maskb
        r = jnp.dot(w, g2b_s[...], preferred_element_type=jnp.float32)
        denom = r[:, F2:F2 + 1]
        gmean = gsum2_s[...] * (1.0 / N)
        out_ref[...] = jnp.where(denom > 0, r[:, :F2] / denom, gmean)


def kernel(x, adj_mat, W1, a1_l, a1_r, W2, a2_l, a2_r):
    adj = adj_mat.reshape(N, N).astype(jnp.int8)
    eye = jnp.eye(H1, dtype=jnp.float32)
    al1 = (eye[:, None, :] * a1_l[None, :, None]).reshape(D1, H1)
    ar1 = (eye[:, None, :] * a1_r[None, :, None]).reshape(D1, H1)
    a2l = a2_l.reshape(F2, 1)
    a2r = a2_r.reshape(F2, 1)

    nb = N // BI
    out = pl.pallas_call(
        _body,
        grid=(2, nb),
        in_specs=[
            pl.BlockSpec((N, x.shape[1]), lambda p, i: (0, 0)),
            pl.BlockSpec((BI, N), lambda p, i: (i, 0)),
            pl.BlockSpec(W1.shape, lambda p, i: (0, 0)),
            pl.BlockSpec((D1, H1), lambda p, i: (0, 0)),
            pl.BlockSpec((D1, H1), lambda p, i: (0, 0)),
            pl.BlockSpec(W2.shape, lambda p, i: (0, 0)),
            pl.BlockSpec((F2, 1), lambda p, i: (0, 0)),
            pl.BlockSpec((F2, 1), lambda p, i: (0, 0)),
        ],
        out_specs=pl.BlockSpec((BI, F2), lambda p, i: (i, 0)),
        out_shape=jax.ShapeDtypeStruct((N, F2), jnp.float32),
        scratch_shapes=[
            pltpu.VMEM((N, D1), jnp.bfloat16),
            pltpu.VMEM((N, H1), jnp.float32),
            pltpu.VMEM((H1, 1), jnp.float32),
            pltpu.VMEM((H1, N), jnp.bfloat16),
            pltpu.VMEM((H1, N), jnp.bfloat16),
            pltpu.VMEM((1, D1), jnp.float32),
            pltpu.VMEM((N, F2 + 1), jnp.bfloat16),
            pltpu.VMEM((N, 1), jnp.float32),
            pltpu.VMEM((N, 1), jnp.float32),
            pltpu.VMEM((1, F2), jnp.float32),
        ],
    )(x, adj, W1, al1, ar1, W2, a2l, a2r)
    return out
